# baseline (speedup 1.0000x reference)
import functools

import jax
import jax.numpy as jnp
from jax import lax
from jax.experimental import pallas as pl
from jax.experimental.pallas import tpu as pltpu
from jax.experimental.pallas import tpu_sc as plsc

NC = 2
NS = 16
NW = NC * NS

VC = 384
VMAIN = 999936
CTOK = 256
PAD_D = 128


def _transpose_body(wt_hbm, tail_hbm, pad_hbm, a_v, b_v, t_v, sem0, sem1):
    d = wt_hbm.shape[0]
    n_chunks = VMAIN // VC
    wid = lax.axis_index("s") * NC + lax.axis_index("c")
    sems = (sem0, sem1)
    iters = n_chunks // NW + 1
    row_iota = lax.iota(jnp.int32, 16)

    def fire(g, buf):
        cid = wid + g * NW

        @pl.when(cid < n_chunks)
        def _():
            for i in range(3):
                pltpu.async_copy(
                    wt_hbm.at[:, pl.ds(cid * VC + i * 128, 128)],
                    a_v.at[buf].at[i],
                    sems[buf],
                )

    def drain(g, buf):
        cid = wid + g * NW

        @pl.when(cid < n_chunks)
        def _():
            for i in range(3):
                pltpu.make_async_copy(
                    wt_hbm.at[:, pl.ds(cid * VC + i * 128, 128)],
                    a_v.at[buf].at[i],
                    sems[buf],
                ).wait()

    fire(0, 0)

    def step(g2, carry):
        for buf in range(2):
            g = g2 * 2 + buf
            fire(g + 1, 1 - buf)
            drain(g, buf)
            cid = wid + g * NW

            @pl.when(cid < n_chunks)
            def _():
                def grp(rg, c2):
                    for i in range(3):
                        rows = row_iota + (i * 128 + rg * 16)
                        for dd in range(d):
                            vals = a_v[buf, i, dd, pl.ds(rg * 16, 16)]
                            plsc.store_scatter(
                                b_v,
                                [rows, jnp.full((16,), dd, jnp.int32)],
                                vals,
                            )
                    return c2

                lax.fori_loop(0, 8, grp, 0)
                pltpu.sync_copy(b_v, pad_hbm.at[pl.ds(cid * VC, VC)])

        return carry

    lax.fori_loop(0, (iters + 1) // 2, step, 0)

    @pl.when(wid == NW - 1)
    def _():
        pltpu.sync_copy(tail_hbm, t_v)
        for rg in range(4):
            rows = row_iota + rg * 16
            for dd in range(d):
                vals = t_v[dd, pl.ds(rg * 16, 16)]
                plsc.store_scatter(
                    b_v, [rows, jnp.full((16,), dd, jnp.int32)], vals
                )
        pltpu.sync_copy(
            b_v.at[pl.ds(0, 64)],
            pad_hbm.at[pl.ds(VMAIN, 1000000 - VMAIN)],
        )


def _gather_body(idx_hbm, pad_hbm, out_hbm, idx_v, rows_v, sem0, sem1):
    n_idx_rows = idx_hbm.shape[0]
    wid = lax.axis_index("s") * NC + lax.axis_index("c")
    rows_per_w = n_idx_rows // NW
    iters = rows_per_w // 2
    r_lo = wid * rows_per_w
    sems = (sem0, sem1)

    def fire(g, b):
        pltpu.sync_copy(idx_hbm.at[pl.ds(r_lo + g * 2, 2)], idx_v.at[b])
        for i in range(2):
            pltpu.async_copy(
                pad_hbm.at[idx_v.at[b].at[i]],
                rows_v.at[b].at[pl.ds(i * 128, 128)],
                sems[b],
            )

    def drain(b):
        for i in range(2):
            pltpu.make_async_copy(
                pad_hbm.at[idx_v.at[b].at[i]],
                rows_v.at[b].at[pl.ds(i * 128, 128)],
                sems[b],
            ).wait()

    fire(0, 0)

    def step(g2, carry):
        for b in range(2):
            g = g2 * 2 + b
            nxt = 1 - b

            @pl.when(g + 1 < iters)
            def _():
                fire(g + 1, nxt)

            drain(b)
            pltpu.sync_copy(
                rows_v.at[b],
                out_hbm.at[pl.ds((r_lo + g * 2) * 128, CTOK)],
            )
        return carry

    lax.fori_loop(0, iters // 2, step, 0)


def _mesh():
    return plsc.VectorSubcoreMesh(
        core_axis_name="c", subcore_axis_name="s", num_cores=NC, num_subcores=NS
    )


_SC_PARAMS = pltpu.CompilerParams(
    use_tc_tiling_on_sc=True, needs_layout_passes=False
)


def _transpose_call(wt, tail128):
    return pl.kernel(
        _transpose_body,
        out_type=jax.ShapeDtypeStruct((1000000, PAD_D), jnp.float32),
        mesh=_mesh(),
        scratch_types=[
            pltpu.VMEM((2, 3, 64, 128), jnp.float32),
            pltpu.VMEM((VC, PAD_D), jnp.float32),
            pltpu.VMEM((64, 128), jnp.float32),
            pltpu.SemaphoreType.DMA,
            pltpu.SemaphoreType.DMA,
        ],
        compiler_params=_SC_PARAMS,
    )(wt, tail128)


def _gather_call(idx2d, pad_table):
    n_tok = idx2d.shape[0] * idx2d.shape[1]
    return pl.kernel(
        _gather_body,
        out_type=jax.ShapeDtypeStruct((n_tok, PAD_D), jnp.float32),
        mesh=_mesh(),
        scratch_types=[
            pltpu.VMEM((2, 2, 128), jnp.int32),
            pltpu.VMEM((2, CTOK, PAD_D), jnp.float32),
            pltpu.SemaphoreType.DMA,
            pltpu.SemaphoreType.DMA,
        ],
        compiler_params=_SC_PARAMS,
    )(idx2d, pad_table)


def kernel(token_ids, weight):
    b, l = token_ids.shape
    wt = weight.T
    tail128 = jnp.pad(
        lax.slice(wt, (0, VMAIN), (64, 1000000)), ((0, 0), (0, 64))
    )
    pad_table = _transpose_call(wt, tail128)
    idx2d = token_ids.astype(jnp.int32).reshape(b * l // 128, 128)
    o128 = _gather_call(idx2d, pad_table)
    return o128[:, :64].reshape(b, l, 64)


# speedup vs baseline: 1.0272x; 1.0272x over previous
import functools

import jax
import jax.numpy as jnp
from jax import lax
from jax.experimental import pallas as pl
from jax.experimental.pallas import tpu as pltpu
from jax.experimental.pallas import tpu_sc as plsc

NC = 2
NS = 16
NW = NC * NS

VC = 384
VMAIN = 999936
CTOK = 256
PAD_D = 128


def _transpose_body(wt_hbm, tail_hbm, pad_hbm, a_v, b_v, t_v, sem0, sem1):
    d = wt_hbm.shape[0]
    n_chunks = VMAIN // VC
    wid = lax.axis_index("s") * NC + lax.axis_index("c")
    sems = (sem0, sem1)
    iters = n_chunks // NW + 1
    row_iota = lax.iota(jnp.int32, 16)

    def fire(g, buf):
        cid = wid + g * NW

        @pl.when(cid < n_chunks)
        def _():
            for i in range(3):
                pltpu.async_copy(
                    wt_hbm.at[:, pl.ds(cid * VC + i * 128, 128)],
                    a_v.at[buf].at[i],
                    sems[buf],
                )

    def drain(g, buf):
        cid = wid + g * NW

        @pl.when(cid < n_chunks)
        def _():
            for i in range(3):
                pltpu.make_async_copy(
                    wt_hbm.at[:, pl.ds(cid * VC + i * 128, 128)],
                    a_v.at[buf].at[i],
                    sems[buf],
                ).wait()

    fire(0, 0)

    def step(g2, carry):
        for buf in range(2):
            g = g2 * 2 + buf
            fire(g + 1, 1 - buf)
            drain(g, buf)
            cid = wid + g * NW

            @pl.when(cid < n_chunks)
            def _():
                @plsc.parallel_loop(0, 8, unroll=2)
                def grp(rg):
                    for i in range(3):
                        rows = row_iota + (i * 128 + rg * 16)
                        for dd in range(d):
                            vals = a_v[buf, i, dd, pl.ds(rg * 16, 16)]
                            plsc.store_scatter(
                                b_v,
                                [rows, jnp.full((16,), dd, jnp.int32)],
                                vals,
                            )

                pltpu.sync_copy(b_v, pad_hbm.at[pl.ds(cid * VC, VC)])

        return carry

    lax.fori_loop(0, (iters + 1) // 2, step, 0)

    @pl.when(wid == NW - 1)
    def _():
        pltpu.sync_copy(tail_hbm, t_v)
        for rg in range(4):
            rows = row_iota + rg * 16
            for dd in range(d):
                vals = t_v[dd, pl.ds(rg * 16, 16)]
                plsc.store_scatter(
                    b_v, [rows, jnp.full((16,), dd, jnp.int32)], vals
                )
        pltpu.sync_copy(
            b_v.at[pl.ds(0, 64)],
            pad_hbm.at[pl.ds(VMAIN, 1000000 - VMAIN)],
        )


def _gather_body(idx_hbm, pad_hbm, out_hbm, idx_v, rows_v, sem0, sem1):
    n_idx_rows = idx_hbm.shape[0]
    wid = lax.axis_index("s") * NC + lax.axis_index("c")
    rows_per_w = n_idx_rows // NW
    iters = rows_per_w // 2
    r_lo = wid * rows_per_w
    sems = (sem0, sem1)

    def fire(g, b):
        pltpu.sync_copy(idx_hbm.at[pl.ds(r_lo + g * 2, 2)], idx_v.at[b])
        for i in range(2):
            pltpu.async_copy(
                pad_hbm.at[idx_v.at[b].at[i]],
                rows_v.at[b].at[pl.ds(i * 128, 128)],
                sems[b],
            )

    def drain(b):
        for i in range(2):
            pltpu.make_async_copy(
                pad_hbm.at[idx_v.at[b].at[i]],
                rows_v.at[b].at[pl.ds(i * 128, 128)],
                sems[b],
            ).wait()

    fire(0, 0)

    def step(g2, carry):
        for b in range(2):
            g = g2 * 2 + b
            nxt = 1 - b

            @pl.when(g + 1 < iters)
            def _():
                fire(g + 1, nxt)

            drain(b)
            pltpu.sync_copy(
                rows_v.at[b],
                out_hbm.at[pl.ds((r_lo + g * 2) * 128, CTOK)],
            )
        return carry

    lax.fori_loop(0, iters // 2, step, 0)


def _mesh():
    return plsc.VectorSubcoreMesh(
        core_axis_name="c", subcore_axis_name="s", num_cores=NC, num_subcores=NS
    )


_SC_PARAMS = pltpu.CompilerParams(
    use_tc_tiling_on_sc=True, needs_layout_passes=False
)


def _transpose_call(wt, tail128):
    return pl.kernel(
        _transpose_body,
        out_type=jax.ShapeDtypeStruct((1000000, PAD_D), jnp.float32),
        mesh=_mesh(),
        scratch_types=[
            pltpu.VMEM((2, 3, 64, 128), jnp.float32),
            pltpu.VMEM((VC, PAD_D), jnp.float32),
            pltpu.VMEM((64, 128), jnp.float32),
            pltpu.SemaphoreType.DMA,
            pltpu.SemaphoreType.DMA,
        ],
        compiler_params=_SC_PARAMS,
    )(wt, tail128)


def _gather_call(idx2d, pad_table):
    n_tok = idx2d.shape[0] * idx2d.shape[1]
    return pl.kernel(
        _gather_body,
        out_type=jax.ShapeDtypeStruct((n_tok, PAD_D), jnp.float32),
        mesh=_mesh(),
        scratch_types=[
            pltpu.VMEM((2, 2, 128), jnp.int32),
            pltpu.VMEM((2, CTOK, PAD_D), jnp.float32),
            pltpu.SemaphoreType.DMA,
            pltpu.SemaphoreType.DMA,
        ],
        compiler_params=_SC_PARAMS,
    )(idx2d, pad_table)


def kernel(token_ids, weight):
    b, l = token_ids.shape
    wt = weight.T
    tail128 = jnp.pad(
        lax.slice(wt, (0, VMAIN), (64, 1000000)), ((0, 0), (0, 64))
    )
    pad_table = _transpose_call(wt, tail128)
    idx2d = token_ids.astype(jnp.int32).reshape(b * l // 128, 128)
    o128 = _gather_call(idx2d, pad_table)
    return o128[:, :64].reshape(b, l, 64)


# speedup vs baseline: 1.9338x; 1.8826x over previous
import functools

import jax
import jax.numpy as jnp
from jax import lax
from jax.experimental import pallas as pl
from jax.experimental.pallas import tpu as pltpu
from jax.experimental.pallas import tpu_sc as plsc

NC = 2
NS = 16
NW = NC * NS

CTOK = 256
PAD_D = 128
TBLK = 2048


def _tc_transpose_body(wt_ref, out_ref):
    out_ref[:, 0:64] = wt_ref[...].T


def _tc_transpose(wt):
    d, v = wt.shape
    grid = (v + TBLK - 1) // TBLK
    return pl.pallas_call(
        _tc_transpose_body,
        out_shape=jax.ShapeDtypeStruct((v, PAD_D), jnp.float32),
        grid=(grid,),
        in_specs=[pl.BlockSpec((d, TBLK), lambda i: (0, i))],
        out_specs=pl.BlockSpec((TBLK, PAD_D), lambda i: (i, 0)),
    )(wt)


def _gather_body(idx_hbm, pad_hbm, out_hbm, idx_v, rows_v, sem0, sem1):
    n_idx_rows = idx_hbm.shape[0]
    wid = lax.axis_index("s") * NC + lax.axis_index("c")
    rows_per_w = n_idx_rows // NW
    iters = rows_per_w // 2
    r_lo = wid * rows_per_w
    sems = (sem0, sem1)

    def fire(g, b):
        pltpu.sync_copy(idx_hbm.at[pl.ds(r_lo + g * 2, 2)], idx_v.at[b])
        for i in range(2):
            pltpu.async_copy(
                pad_hbm.at[idx_v.at[b].at[i]],
                rows_v.at[b].at[pl.ds(i * 128, 128)],
                sems[b],
            )

    def drain(b):
        for i in range(2):
            pltpu.make_async_copy(
                pad_hbm.at[idx_v.at[b].at[i]],
                rows_v.at[b].at[pl.ds(i * 128, 128)],
                sems[b],
            ).wait()

    fire(0, 0)

    def step(g2, carry):
        for b in range(2):
            g = g2 * 2 + b
            nxt = 1 - b

            @pl.when(g + 1 < iters)
            def _():
                fire(g + 1, nxt)

            drain(b)
            pltpu.sync_copy(
                rows_v.at[b],
                out_hbm.at[pl.ds((r_lo + g * 2) * 128, CTOK)],
            )
        return carry

    lax.fori_loop(0, iters // 2, step, 0)


def _gather_call(idx2d, pad_table):
    n_tok = idx2d.shape[0] * idx2d.shape[1]
    mesh = plsc.VectorSubcoreMesh(
        core_axis_name="c", subcore_axis_name="s", num_cores=NC, num_subcores=NS
    )
    return pl.kernel(
        _gather_body,
        out_type=jax.ShapeDtypeStruct((n_tok, PAD_D), jnp.float32),
        mesh=mesh,
        scratch_types=[
            pltpu.VMEM((2, 2, 128), jnp.int32),
            pltpu.VMEM((2, CTOK, PAD_D), jnp.float32),
            pltpu.SemaphoreType.DMA,
            pltpu.SemaphoreType.DMA,
        ],
        compiler_params=pltpu.CompilerParams(
            use_tc_tiling_on_sc=True, needs_layout_passes=False
        ),
    )(idx2d, pad_table)


def kernel(token_ids, weight):
    b, l = token_ids.shape
    wt = weight.T
    pad_table = _tc_transpose(wt)
    idx2d = token_ids.astype(jnp.int32).reshape(b * l // 128, 128)
    o128 = _gather_call(idx2d, pad_table)
    return o128[:, :64].reshape(b, l, 64)


# speedup vs baseline: 2.4405x; 1.2620x over previous
import functools

import jax
import jax.numpy as jnp
from jax import lax
from jax.experimental import pallas as pl
from jax.experimental.pallas import tpu as pltpu
from jax.experimental.pallas import tpu_sc as plsc

NC = 2
NS = 16
NW = NC * NS

CTOK = 256
PAD_D = 128
TBLK = 8192


def _tc_transpose_body(wt_ref, out_ref):
    out_ref[:, 0:64] = wt_ref[...].T


def _tc_transpose(wt):
    d, v = wt.shape
    grid = (v + TBLK - 1) // TBLK
    return pl.pallas_call(
        _tc_transpose_body,
        out_shape=jax.ShapeDtypeStruct((v, PAD_D), jnp.float32),
        grid=(grid,),
        in_specs=[pl.BlockSpec((d, TBLK), lambda i: (0, i))],
        out_specs=pl.BlockSpec((TBLK, PAD_D), lambda i: (i, 0)),
    )(wt)


def _gather_body(idx_hbm, pad_hbm, out_hbm, idx_v, rows_v, sem0, sem1):
    n_idx_rows = idx_hbm.shape[0]
    wid = lax.axis_index("s") * NC + lax.axis_index("c")
    rows_per_w = n_idx_rows // NW
    iters = rows_per_w // 2
    r_lo = wid * rows_per_w
    sems = (sem0, sem1)

    def fire(g, b):
        pltpu.sync_copy(idx_hbm.at[pl.ds(r_lo + g * 2, 2)], idx_v.at[b])
        for i in range(2):
            pltpu.async_copy(
                pad_hbm.at[idx_v.at[b].at[i]],
                rows_v.at[b].at[pl.ds(i * 128, 128)],
                sems[b],
            )

    def drain(b):
        for i in range(2):
            pltpu.make_async_copy(
                pad_hbm.at[idx_v.at[b].at[i]],
                rows_v.at[b].at[pl.ds(i * 128, 128)],
                sems[b],
            ).wait()

    fire(0, 0)

    def step(g2, carry):
        for b in range(2):
            g = g2 * 2 + b
            nxt = 1 - b

            @pl.when(g + 1 < iters)
            def _():
                fire(g + 1, nxt)

            drain(b)
            pltpu.sync_copy(
                rows_v.at[b],
                out_hbm.at[pl.ds((r_lo + g * 2) * 128, CTOK)],
            )
        return carry

    lax.fori_loop(0, iters // 2, step, 0)


def _gather_call(idx2d, pad_table):
    n_tok = idx2d.shape[0] * idx2d.shape[1]
    mesh = plsc.VectorSubcoreMesh(
        core_axis_name="c", subcore_axis_name="s", num_cores=NC, num_subcores=NS
    )
    return pl.kernel(
        _gather_body,
        out_type=jax.ShapeDtypeStruct((n_tok, PAD_D), jnp.float32),
        mesh=mesh,
        scratch_types=[
            pltpu.VMEM((2, 2, 128), jnp.int32),
            pltpu.VMEM((2, CTOK, PAD_D), jnp.float32),
            pltpu.SemaphoreType.DMA,
            pltpu.SemaphoreType.DMA,
        ],
        compiler_params=pltpu.CompilerParams(
            use_tc_tiling_on_sc=True, needs_layout_passes=False
        ),
    )(idx2d, pad_table)


def kernel(token_ids, weight):
    b, l = token_ids.shape
    wt = weight.T
    pad_table = _tc_transpose(wt)
    idx2d = token_ids.astype(jnp.int32).reshape(b * l // 128, 128)
    o128 = _gather_call(idx2d, pad_table)
    return o128[:, :64].reshape(b, l, 64)


# speedup vs baseline: 2.5027x; 1.0255x over previous
import functools

import jax
import jax.numpy as jnp
from jax import lax
from jax.experimental import pallas as pl
from jax.experimental.pallas import tpu as pltpu
from jax.experimental.pallas import tpu_sc as plsc

NC = 2
NS = 16
NW = NC * NS

CTOK = 256
PAD_D = 128
TBLK = 16384


def _tc_transpose_body(wt_ref, out_ref):
    out_ref[:, 0:64] = wt_ref[...].T


def _tc_transpose(wt):
    d, v = wt.shape
    grid = (v + TBLK - 1) // TBLK
    return pl.pallas_call(
        _tc_transpose_body,
        out_shape=jax.ShapeDtypeStruct((v, PAD_D), jnp.float32),
        grid=(grid,),
        in_specs=[pl.BlockSpec((d, TBLK), lambda i: (0, i))],
        out_specs=pl.BlockSpec((TBLK, PAD_D), lambda i: (i, 0)),
    )(wt)


def _gather_body(idx_hbm, pad_hbm, out_hbm, idx_v, rows_v, sem0, sem1):
    n_idx_rows = idx_hbm.shape[0]
    wid = lax.axis_index("s") * NC + lax.axis_index("c")
    rows_per_w = n_idx_rows // NW
    iters = rows_per_w // 2
    r_lo = wid * rows_per_w
    sems = (sem0, sem1)

    def fire(g, b):
        pltpu.sync_copy(idx_hbm.at[pl.ds(r_lo + g * 2, 2)], idx_v.at[b])
        for i in range(2):
            pltpu.async_copy(
                pad_hbm.at[idx_v.at[b].at[i]],
                rows_v.at[b].at[pl.ds(i * 128, 128)],
                sems[b],
            )

    def drain(b):
        for i in range(2):
            pltpu.make_async_copy(
                pad_hbm.at[idx_v.at[b].at[i]],
                rows_v.at[b].at[pl.ds(i * 128, 128)],
                sems[b],
            ).wait()

    fire(0, 0)

    def step(g2, carry):
        for b in range(2):
            g = g2 * 2 + b
            nxt = 1 - b

            @pl.when(g + 1 < iters)
            def _():
                fire(g + 1, nxt)

            drain(b)
            pltpu.sync_copy(
                rows_v.at[b],
                out_hbm.at[pl.ds((r_lo + g * 2) * 128, CTOK)],
            )
        return carry

    lax.fori_loop(0, iters // 2, step, 0)


def _gather_call(idx2d, pad_table):
    n_tok = idx2d.shape[0] * idx2d.shape[1]
    mesh = plsc.VectorSubcoreMesh(
        core_axis_name="c", subcore_axis_name="s", num_cores=NC, num_subcores=NS
    )
    return pl.kernel(
        _gather_body,
        out_type=jax.ShapeDtypeStruct((n_tok, PAD_D), jnp.float32),
        mesh=mesh,
        scratch_types=[
            pltpu.VMEM((2, 2, 128), jnp.int32),
            pltpu.VMEM((2, CTOK, PAD_D), jnp.float32),
            pltpu.SemaphoreType.DMA,
            pltpu.SemaphoreType.DMA,
        ],
        compiler_params=pltpu.CompilerParams(
            use_tc_tiling_on_sc=True, needs_layout_passes=False
        ),
    )(idx2d, pad_table)


def kernel(token_ids, weight):
    b, l = token_ids.shape
    wt = weight.T
    pad_table = _tc_transpose(wt)
    idx2d = token_ids.astype(jnp.int32).reshape(b * l // 128, 128)
    o128 = _gather_call(idx2d, pad_table)
    return o128[:, :64].reshape(b, l, 64)
